# Initial kernel scaffold; baseline (speedup 1.0000x reference)
import jax
import jax.numpy as jnp
from jax.experimental import pallas as pl


def kernel(w1, b1, w2, b2, w3, b3, wf1, bf1, wf2, bf2, x):
    raise NotImplementedError("write your pallas kernel here")


# speedup vs baseline: 4.3756x; 4.3756x over previous
import jax
import jax.numpy as jnp
from jax.experimental import pallas as pl
from jax.experimental.pallas import tpu as pltpu

_S1 = 104


def _v2v_kernel(x_ref, w1_ref, b1_ref, w2_ref, b2_ref, w3_ref, b3_ref,
                wf1_ref, bf1_ref, wf2_ref, bf2_ref, out_ref,
                a1_ref, p1_ref, a2_ref, p2_ref, a3_ref):
    f32 = jnp.float32
    tn = out_ref.shape[0]

    def conv_lrelu(in_ref, w_ref, b_ref, out_scr, rows):
        K = w_ref.shape[0]
        cout = w_ref.shape[2]
        ch_max = max(8, 24576 // cout)
        r0 = 0
        while r0 < rows:
            ch = min(ch_max, rows - r0)
            acc = jnp.dot(in_ref[pl.ds(r0, ch), :], w_ref[0],
                          preferred_element_type=f32)
            for k in range(1, K):
                acc = acc + jnp.dot(in_ref[pl.ds(r0 + k, ch), :], w_ref[k],
                                    preferred_element_type=f32)
            z = acc + b_ref[...]
            out_scr[pl.ds(r0, ch), :] = jnp.maximum(z, 0.01 * z)
            r0 += ch

    def maxpool2(in_scr, out_scr):
        m = out_scr.shape[0] - 8
        out_scr[pl.ds(0, m), :] = jnp.maximum(in_scr[pl.ds(0, m, 2), :],
                                              in_scr[pl.ds(1, m, 2), :])
        out_scr[pl.ds(m, 8), :] = jnp.zeros((8, out_scr.shape[1]), f32)

    r1 = a1_ref.shape[0]
    conv_lrelu(x_ref, w1_ref, b1_ref, a1_ref, r1 - 8)
    a1_ref[pl.ds(r1 - 8, 8), :] = jnp.zeros((8, a1_ref.shape[1]), f32)

    maxpool2(a1_ref, p1_ref)
    conv_lrelu(p1_ref, w2_ref, b2_ref, a2_ref, a2_ref.shape[0])
    maxpool2(a2_ref, p2_ref)
    conv_lrelu(p2_ref, w3_ref, b3_ref, a3_ref, a3_ref.shape[0])

    slot3 = a3_ref.shape[0] // tn
    hidden = wf1_ref.shape[2]
    acc = jnp.zeros((tn, hidden), f32)
    for l in range(wf1_ref.shape[0]):
        rows = jnp.maximum(a3_ref[pl.ds(2 * l, tn, slot3), :],
                           a3_ref[pl.ds(2 * l + 1, tn, slot3), :])
        acc = acc + jnp.dot(rows, wf1_ref[l], preferred_element_type=f32)
    f1 = jnp.maximum(acc + bf1_ref[...], 0.0)

    f2 = jnp.tanh(jnp.dot(f1, wf2_ref[...], preferred_element_type=f32)
                  + bf2_ref[...])
    inv = jax.lax.rsqrt(jnp.sum(f2 * f2, axis=-1, keepdims=True) + 1e-12)
    out_ref[...] = f2 * inv


def kernel(w1, b1, w2, b2, w3, b3, wf1, bf1, wf2, bf2, x):
    B, three, F_, T = x.shape
    N = B * three
    dim = wf2.shape[1]
    tile_n = 64
    tn = max(8, (min(tile_n, N) + 7) // 8 * 8)
    n_pad = (N + tn - 1) // tn * tn

    xt = jnp.transpose(x.reshape(N, F_, T), (0, 2, 1)).astype(jnp.float32)
    xt = jnp.pad(xt, ((0, n_pad - N), (0, _S1 - T), (0, 0)))
    x2d = xt.reshape(n_pad * _S1, F_)

    full = lambda a: pl.BlockSpec(a.shape, lambda i, _nd=a.ndim: (0,) * _nd)

    out = pl.pallas_call(
        _v2v_kernel,
        out_shape=jax.ShapeDtypeStruct((n_pad, dim), jnp.float32),
        grid=(n_pad // tn,),
        in_specs=[
            pl.BlockSpec((tn * _S1, F_), lambda i: (i, 0)),
            full(w1), full(b1),
            full(w2), full(b2),
            full(w3), full(b3),
            full(wf1), full(bf1),
            full(wf2), full(bf2),
        ],
        out_specs=pl.BlockSpec((tn, dim), lambda i: (i, 0)),
        scratch_shapes=[
            pltpu.VMEM((tn * _S1, 32), jnp.float32),
            pltpu.VMEM((tn * 52 + 8, 32), jnp.float32),
            pltpu.VMEM((tn * 52, 64), jnp.float32),
            pltpu.VMEM((tn * 26 + 8, 64), jnp.float32),
            pltpu.VMEM((tn * 26, 128), jnp.float32),
        ],
        compiler_params=pltpu.CompilerParams(
            dimension_semantics=("parallel",),
            vmem_limit_bytes=64 * 1024 * 1024),
    )(x2d, w1, b1, w2, b2, w3, b3, wf1, bf1, wf2, bf2)

    return out[:N].reshape(B, three, dim)


# speedup vs baseline: 4.6852x; 1.0708x over previous
import jax
import jax.numpy as jnp
from jax.experimental import pallas as pl
from jax.experimental.pallas import tpu as pltpu

_S1 = 104


def _v2v_kernel(x_ref, w1_ref, b1_ref, w2_ref, b2_ref, w3_ref, b3_ref,
                wf1_ref, bf1_ref, wf2_ref, bf2_ref, out_ref,
                a1_ref, p1_ref, a2_ref, p2_ref, a3_ref):
    f32 = jnp.float32
    bf16 = jnp.bfloat16
    tn = out_ref.shape[0]

    def conv_lrelu(in_ref, w_ref, b_ref, out_scr, rows):
        K = w_ref.shape[0]
        cout = w_ref.shape[2]
        ch_max = max(8, 24576 // cout)
        r0 = 0
        while r0 < rows:
            ch = min(ch_max, rows - r0)
            acc = jnp.dot(in_ref[pl.ds(r0, ch), :], w_ref[0],
                          preferred_element_type=f32)
            for k in range(1, K):
                acc = acc + jnp.dot(in_ref[pl.ds(r0 + k, ch), :], w_ref[k],
                                    preferred_element_type=f32)
            z = acc + b_ref[...]
            out_scr[pl.ds(r0, ch), :] = jnp.maximum(z, 0.01 * z)
            r0 += ch

    def maxpool2(in_scr, out_scr):
        m = out_scr.shape[0] - 8
        mx = jnp.maximum(in_scr[pl.ds(0, m, 2), :], in_scr[pl.ds(1, m, 2), :])
        out_scr[pl.ds(0, m), :] = mx.astype(bf16)
        out_scr[pl.ds(m, 8), :] = jnp.zeros((8, out_scr.shape[1]), bf16)

    r1 = a1_ref.shape[0]
    conv_lrelu(x_ref, w1_ref, b1_ref, a1_ref, r1 - 8)
    a1_ref[pl.ds(r1 - 8, 8), :] = jnp.zeros((8, a1_ref.shape[1]), f32)

    maxpool2(a1_ref, p1_ref)
    conv_lrelu(p1_ref, w2_ref, b2_ref, a2_ref, a2_ref.shape[0])
    maxpool2(a2_ref, p2_ref)
    conv_lrelu(p2_ref, w3_ref, b3_ref, a3_ref, a3_ref.shape[0])

    slot3 = a3_ref.shape[0] // tn
    hidden = wf1_ref.shape[2]
    acc = jnp.zeros((tn, hidden), f32)
    for l in range(wf1_ref.shape[0]):
        rows = jnp.maximum(a3_ref[pl.ds(2 * l, tn, slot3), :],
                           a3_ref[pl.ds(2 * l + 1, tn, slot3), :])
        acc = acc + jnp.dot(rows.astype(bf16), wf1_ref[l],
                            preferred_element_type=f32)
    f1 = jnp.maximum(acc + bf1_ref[...], 0.0).astype(bf16)

    f2 = jnp.tanh(jnp.dot(f1, wf2_ref[...], preferred_element_type=f32)
                  + bf2_ref[...])
    inv = jax.lax.rsqrt(jnp.sum(f2 * f2, axis=-1, keepdims=True) + 1e-12)
    out_ref[...] = f2 * inv


def kernel(w1, b1, w2, b2, w3, b3, wf1, bf1, wf2, bf2, x):
    B, three, F_, T = x.shape
    N = B * three
    dim = wf2.shape[1]
    tile_n = 64
    tn = max(8, (min(tile_n, N) + 7) // 8 * 8)
    n_pad = (N + tn - 1) // tn * tn

    xt = jnp.transpose(x.reshape(N, F_, T), (0, 2, 1)).astype(jnp.bfloat16)
    xt = jnp.pad(xt, ((0, n_pad - N), (0, _S1 - T), (0, 0)))
    x2d = xt.reshape(n_pad * _S1, F_)
    w1b = w1.astype(jnp.bfloat16)
    w2b = w2.astype(jnp.bfloat16)
    w3b = w3.astype(jnp.bfloat16)
    wf1b = wf1.astype(jnp.bfloat16)
    wf2b = wf2.astype(jnp.bfloat16)

    full = lambda a: pl.BlockSpec(a.shape, lambda i, _nd=a.ndim: (0,) * _nd)

    out = pl.pallas_call(
        _v2v_kernel,
        out_shape=jax.ShapeDtypeStruct((n_pad, dim), jnp.float32),
        grid=(n_pad // tn,),
        in_specs=[
            pl.BlockSpec((tn * _S1, F_), lambda i: (i, 0)),
            full(w1b), full(b1),
            full(w2b), full(b2),
            full(w3b), full(b3),
            full(wf1b), full(bf1),
            full(wf2b), full(bf2),
        ],
        out_specs=pl.BlockSpec((tn, dim), lambda i: (i, 0)),
        scratch_shapes=[
            pltpu.VMEM((tn * _S1, 32), jnp.float32),
            pltpu.VMEM((tn * 52 + 8, 32), jnp.bfloat16),
            pltpu.VMEM((tn * 52, 64), jnp.float32),
            pltpu.VMEM((tn * 26 + 8, 64), jnp.bfloat16),
            pltpu.VMEM((tn * 26, 128), jnp.float32),
        ],
        compiler_params=pltpu.CompilerParams(
            dimension_semantics=("parallel",),
            vmem_limit_bytes=64 * 1024 * 1024),
    )(x2d, w1b, b1, w2b, b2, w3b, b3, wf1b, bf1, wf2b, bf2)

    return out[:N].reshape(B, three, dim)
